# baseline (speedup 1.0000x reference)
import functools

import jax
import jax.numpy as jnp
import numpy as np
from jax.experimental import pallas as pl
from jax.experimental.pallas import tpu as pltpu

_TINY = np.float32(1.1754943508222875e-38)


def _rotl(x, r):
    return (x << jnp.uint32(r)) | (x >> jnp.uint32(32 - r))


def _round4(x0, x1, rots):
    for r in rots:
        x0 = x0 + x1
        x1 = _rotl(x1, r) ^ x0
    return x0, x1


def _threefry_bits(x_lo):
    ks0 = jnp.uint32(0)
    ks1 = jnp.uint32(42)
    ks2 = jnp.uint32(0x1BD11BDA ^ 42)
    rot_a = (13, 15, 26, 6)
    rot_b = (17, 29, 16, 24)

    x1 = x_lo + ks1
    x0 = x1
    x1 = _rotl(x1, 13) ^ x0
    x0, x1 = _round4(x0, x1, rot_a[1:])
    x0 = x0 + ks1
    x1 = x1 + (ks2 + jnp.uint32(1))
    x0, x1 = _round4(x0, x1, rot_b)
    x0 = x0 + ks2
    x1 = x1 + jnp.uint32(2)
    x0, x1 = _round4(x0, x1, rot_a)
    x1 = x1 + (ks1 + jnp.uint32(3))
    x0, x1 = _round4(x0, x1, rot_b)
    x0 = x0 + ks1
    x1 = x1 + (ks2 + jnp.uint32(4))
    x0, x1 = _round4(x0, x1, rot_a)
    x0 = x0 + ks2
    x1 = x1 + jnp.uint32(5)
    return x0 ^ x1


def _body(vocab, n_chunks, lg_ref, out_ref, mx_ref, ix_ref):
    k = pl.program_id(1)
    rblk = pl.program_id(0)
    nb, nc = lg_ref.shape

    row = jax.lax.broadcasted_iota(jnp.uint32, (nb, nc), 0)
    col = jax.lax.broadcasted_iota(jnp.uint32, (nb, nc), 1)
    row0 = (jnp.uint32(rblk) * jnp.uint32(nb)).astype(jnp.uint32)
    col0 = jnp.uint32(k) * jnp.uint32(nc)
    x_lo = (row + row0) * jnp.uint32(vocab) + (col + col0)

    bits = _threefry_bits(x_lo)
    fb = (bits >> jnp.uint32(9)) | jnp.uint32(0x3F800000)
    u = jax.lax.bitcast_convert_type(fb, jnp.float32) - jnp.float32(1.0)
    uni = jnp.maximum(u, _TINY)
    g = -jnp.log(-jnp.log(uni))
    score = g + lg_ref[...]

    gcol = col.astype(jnp.int32) + (k * nc)
    score = jnp.where(gcol < vocab, score, -jnp.inf)

    m = jnp.max(score, axis=1, keepdims=True)
    li = jnp.argmax(score, axis=1).astype(jnp.int32).reshape(nb, 1)
    gi = li + (k * nc)

    @pl.when(k == 0)
    def _init():
        mx_ref[...] = m
        ix_ref[...] = gi

    @pl.when(k > 0)
    def _merge():
        better = m > mx_ref[...]
        mx_ref[...] = jnp.where(better, m, mx_ref[...])
        ix_ref[...] = jnp.where(better, gi, ix_ref[...])

    @pl.when(k == n_chunks - 1)
    def _emit():
        out_ref[...] = ix_ref[...]


def kernel(logits):
    batch, vocab = logits.shape
    row_blk = 16
    chunk = 8192
    n_rblk = pl.cdiv(batch, row_blk)
    n_chunks = pl.cdiv(vocab, chunk)

    out = pl.pallas_call(
        functools.partial(_body, vocab, n_chunks),
        grid=(n_rblk, n_chunks),
        in_specs=[pl.BlockSpec((row_blk, chunk), lambda r, k: (r, k))],
        out_specs=pl.BlockSpec((row_blk, 1), lambda r, k: (r, 0)),
        out_shape=jax.ShapeDtypeStruct((batch, 1), jnp.int32),
        scratch_shapes=[
            pltpu.VMEM((row_blk, 1), jnp.float32),
            pltpu.VMEM((row_blk, 1), jnp.int32),
        ],
        compiler_params=pltpu.CompilerParams(
            dimension_semantics=("parallel", "arbitrary"),
        ),
    )(logits)
    return out.reshape(batch)
